# baseline (speedup 1.0000x reference)
import functools

import jax
import jax.numpy as jnp
from jax import lax
from jax.experimental import pallas as pl
from jax.experimental.pallas import tpu as pltpu
from jax.experimental.pallas import tpu_sc as plsc

N = 10000
E = 320000
DIN = 128
DIM = 128
DOUT = 16

NPAD = 10240
CHUNK = 128
NWORK = 32
CPT = 80
EPAD = NWORK * CPT * CHUNK
BN_INV = 0.9999950000374996


def _pad_edges(edge_index):
    npad = EPAD - E
    ar = jnp.arange(npad, dtype=jnp.int32)
    pad = jnp.stack([(ar * 37) % N, N + (ar % (NPAD - N))])
    return jnp.concatenate([edge_index, pad], axis=1).reshape(2, -1, CHUNK)


def _sc_degree(edge3):
    info = plsc.get_sparse_core_info()
    NC, NS = info.num_cores, info.num_subcores
    mesh = plsc.VectorSubcoreMesh(core_axis_name="c", subcore_axis_name="s")
    rows_per_tile = NPAD // NS

    @functools.partial(
        pl.kernel, mesh=mesh,
        out_type=jax.ShapeDtypeStruct((NC, NPAD), jnp.float32),
        scratch_types=[
            pltpu.VMEM((CPT, CHUNK), jnp.int32),
            pltpu.VMEM((CHUNK,), jnp.float32),
            pltpu.VMEM((rows_per_tile,), jnp.float32),
            pltpu.VMEM_SHARED((NPAD,), jnp.float32),
            pltpu.SemaphoreType.DMA,
        ],
    )
    def k(edge_hbm, out_hbm, col_v, ones_v, zero_v, acc, ssem):
        c = lax.axis_index("c")
        s = lax.axis_index("s")
        wid = c * NS + s
        for j in range(CHUNK // 16):
            ones_v[pl.ds(j * 16, 16)] = jnp.full((16,), 1.0, jnp.float32)

        def zbody(i, _):
            zero_v[pl.ds(i * 16, 16)] = jnp.zeros((16,), jnp.float32)
            return 0
        lax.fori_loop(0, rows_per_tile // 16, zbody, 0)
        pltpu.sync_copy(zero_v, acc.at[pl.ds(s * rows_per_tile, rows_per_tile)])
        pltpu.sync_copy(edge_hbm.at[1, pl.ds(wid * CPT, CPT), :], col_v)
        plsc.subcore_barrier()

        def body(i, _):
            pltpu.async_copy(ones_v, acc.at[col_v.at[i]], ssem, add=True)
            return 0
        lax.fori_loop(0, CPT, body, 0)

        def drain(i, _):
            pltpu.make_async_copy(ones_v, acc.at[col_v.at[0]], ssem).wait()
            return 0
        lax.fori_loop(0, CPT, drain, 0)
        plsc.subcore_barrier()
        pltpu.sync_copy(acc.at[pl.ds(s * rows_per_tile, rows_per_tile)],
                        out_hbm.at[c, pl.ds(s * rows_per_tile, rows_per_tile)])

    return k(edge3)


def _sc_aggregate(edge3, y):
    info = plsc.get_sparse_core_info()
    NC, NS = info.num_cores, info.num_subcores
    mesh = plsc.VectorSubcoreMesh(core_axis_name="c", subcore_axis_name="s")
    rows_per_tile = NPAD // NS

    IB = 16

    @functools.partial(
        pl.kernel, mesh=mesh,
        out_type=jax.ShapeDtypeStruct((NC, NPAD, DIM), jnp.float32),
        scratch_types=[
            pltpu.VMEM((IB, CHUNK), jnp.int32),
            pltpu.VMEM((IB, CHUNK), jnp.int32),
            pltpu.VMEM((CHUNK, DIM), jnp.float32),
            pltpu.VMEM((CHUNK, DIM), jnp.float32),
            pltpu.VMEM_SHARED((NPAD, DIM), jnp.float32),
            pltpu.SemaphoreType.DMA,
            pltpu.SemaphoreType.DMA,
            pltpu.SemaphoreType.DMA,
            pltpu.SemaphoreType.DMA,
        ],
    )
    def k(edge_hbm, y_hbm, out_hbm, row_v, col_v, buf0, buf1, acc,
          g0, g1, s0, s1):
        c = lax.axis_index("c")
        s = lax.axis_index("s")
        wid = c * NS + s

        def zbody(r, _):
            for j in range(DIM // 16):
                buf0[r, pl.ds(j * 16, 16)] = jnp.zeros((16,), jnp.float32)
            return 0
        lax.fori_loop(0, CHUNK, zbody, 0)
        for b in range(rows_per_tile // CHUNK):
            pltpu.sync_copy(
                buf0,
                acc.at[pl.ds(s * rows_per_tile + b * CHUNK, CHUNK), :])
        plsc.subcore_barrier()

        def slab(sb, _):
            base = wid * CPT + sb * IB
            pltpu.sync_copy(edge_hbm.at[0, pl.ds(base, IB), :], row_v)
            pltpu.sync_copy(edge_hbm.at[1, pl.ds(base, IB), :], col_v)
            pltpu.async_copy(y_hbm.at[row_v.at[0]], buf0, g0)
            pltpu.async_copy(y_hbm.at[row_v.at[1]], buf1, g1)

            def body(i, _):
                ka = 2 * i
                pltpu.make_async_copy(y_hbm.at[row_v.at[ka]], buf0, g0).wait()
                pltpu.async_copy(buf0, acc.at[col_v.at[ka]], s0, add=True)
                pltpu.make_async_copy(
                    y_hbm.at[row_v.at[ka + 1]], buf1, g1).wait()
                pltpu.async_copy(buf1, acc.at[col_v.at[ka + 1]], s1, add=True)
                kn = jnp.minimum(ka + 2, IB - 1)
                km = jnp.minimum(ka + 3, IB - 1)
                pltpu.make_async_copy(buf0, acc.at[col_v.at[0]], s0).wait()
                pltpu.async_copy(y_hbm.at[row_v.at[kn]], buf0, g0)
                pltpu.make_async_copy(buf1, acc.at[col_v.at[0]], s1).wait()
                pltpu.async_copy(y_hbm.at[row_v.at[km]], buf1, g1)
                return 0
            lax.fori_loop(0, IB // 2, body, 0)
            pltpu.make_async_copy(y_hbm.at[row_v.at[0]], buf0, g0).wait()
            pltpu.make_async_copy(y_hbm.at[row_v.at[0]], buf1, g1).wait()
            return 0
        lax.fori_loop(0, CPT // IB, slab, 0)
        plsc.subcore_barrier()
        pltpu.sync_copy(
            acc.at[pl.ds(s * rows_per_tile, rows_per_tile), :],
            out_hbm.at[c, pl.ds(s * rows_per_tile, rows_per_tile), :])

    return k(edge3, y)


_R = 2000


def _dinv(d_ref):
    return lax.rsqrt(d_ref[0] + d_ref[1] + 1.0)


def _dspec():
    return pl.BlockSpec((2, _R, 1), lambda i: (0, i, 0))


def _aspec():
    return pl.BlockSpec((2, _R, DIM), lambda i: (0, i, 0))


def _tc_first(x, W1, deg3):
    def body(x_ref, w_ref, d_ref, y_ref):
        dinv = _dinv(d_ref)
        xl = lax.dot_general(x_ref[...], w_ref[...], (((1,), (1,)), ((), ())),
                             preferred_element_type=jnp.float32)
        y_ref[...] = dinv * xl

    return pl.pallas_call(
        body,
        grid=(N // _R,),
        in_specs=[
            pl.BlockSpec((_R, DIN), lambda i: (i, 0)),
            pl.BlockSpec((DIM, DIN), lambda i: (0, 0)),
            _dspec(),
        ],
        out_specs=pl.BlockSpec((_R, DIM), lambda i: (i, 0)),
        out_shape=jax.ShapeDtypeStruct((N, DIM), jnp.float32),
    )(x, W1, deg3)


def _tc_mid(agg, y1, deg3, b1, g1, be1, W2):
    def body(a_ref, y_ref, d_ref, b_ref, g_ref, be_ref, w_ref, o_ref):
        dinv = _dinv(d_ref)
        pre = dinv * (a_ref[0] + a_ref[1] + y_ref[...]) + b_ref[...]
        h = jnp.maximum(pre, 0.0) * (g_ref[...] * BN_INV) + be_ref[...]
        hl = lax.dot_general(h, w_ref[...], (((1,), (1,)), ((), ())),
                             preferred_element_type=jnp.float32)
        o_ref[...] = dinv * hl

    return pl.pallas_call(
        body,
        grid=(N // _R,),
        in_specs=[
            _aspec(),
            pl.BlockSpec((_R, DIM), lambda i: (i, 0)),
            _dspec(),
            pl.BlockSpec((1, DIM), lambda i: (0, 0)),
            pl.BlockSpec((1, DIM), lambda i: (0, 0)),
            pl.BlockSpec((1, DIM), lambda i: (0, 0)),
            pl.BlockSpec((DIM, DIM), lambda i: (0, 0)),
        ],
        out_specs=pl.BlockSpec((_R, DIM), lambda i: (i, 0)),
        out_shape=jax.ShapeDtypeStruct((N, DIM), jnp.float32),
    )(agg, y1, deg3, b1, g1, be1, W2)


def _tc_final(agg, y2, deg3, b2, g2, be2, l1W, l1b, l2W, l2b):
    def body(a_ref, y_ref, d_ref, b_ref, g_ref, be_ref,
             w1_ref, w1b_ref, w2_ref, w2b_ref, o_ref):
        dinv = _dinv(d_ref)
        pre = dinv * (a_ref[0] + a_ref[1] + y_ref[...]) + b_ref[...]
        h = jnp.maximum(pre, 0.0) * (g_ref[...] * BN_INV) + be_ref[...]
        h = lax.dot_general(h, w1_ref[...], (((1,), (1,)), ((), ())),
                            preferred_element_type=jnp.float32) + w1b_ref[...]
        h = jnp.maximum(h, 0.0)
        lg = lax.dot_general(h, w2_ref[...], (((1,), (1,)), ((), ())),
                             preferred_element_type=jnp.float32) + w2b_ref[...]
        m = jnp.max(lg, axis=-1, keepdims=True)
        lse = jnp.log(jnp.sum(jnp.exp(lg - m), axis=-1, keepdims=True)) + m
        o_ref[...] = lg - lse

    return pl.pallas_call(
        body,
        grid=(N // _R,),
        in_specs=[
            _aspec(),
            pl.BlockSpec((_R, DIM), lambda i: (i, 0)),
            _dspec(),
            pl.BlockSpec((1, DIM), lambda i: (0, 0)),
            pl.BlockSpec((1, DIM), lambda i: (0, 0)),
            pl.BlockSpec((1, DIM), lambda i: (0, 0)),
            pl.BlockSpec((DIM, DIM), lambda i: (0, 0)),
            pl.BlockSpec((1, DIM), lambda i: (0, 0)),
            pl.BlockSpec((DOUT, DIM), lambda i: (0, 0)),
            pl.BlockSpec((1, DOUT), lambda i: (0, 0)),
        ],
        out_specs=pl.BlockSpec((_R, DOUT), lambda i: (i, 0)),
        out_shape=jax.ShapeDtypeStruct((N, DOUT), jnp.float32),
    )(agg, y2, deg3, b2, g2, be2, l1W, l1b, l2W, l2b)


def kernel(x, edge_index, batch, W1, b1, g1, be1, W2, b2, g2, be2,
           l1W, l1b, l2W, l2b):
    del batch
    edge3 = _pad_edges(edge_index.astype(jnp.int32))

    deg3 = _sc_degree(edge3).reshape(2, NPAD, 1)
    y1 = _tc_first(x, W1, deg3)
    agg1 = _sc_aggregate(edge3, y1)
    y2 = _tc_mid(agg1, y1, deg3,
                 b1.reshape(1, DIM), g1.reshape(1, DIM), be1.reshape(1, DIM),
                 W2)
    agg2 = _sc_aggregate(edge3, y2)
    return _tc_final(agg2, y2, deg3,
                     b2.reshape(1, DIM), g2.reshape(1, DIM),
                     be2.reshape(1, DIM),
                     l1W, l1b.reshape(1, DIM), l2W, l2b.reshape(1, DOUT))


# speedup vs baseline: 1.1182x; 1.1182x over previous
import functools

import jax
import jax.numpy as jnp
from jax import lax
from jax.experimental import pallas as pl
from jax.experimental.pallas import tpu as pltpu
from jax.experimental.pallas import tpu_sc as plsc

N = 10000
E = 320000
DIN = 128
DIM = 128
DOUT = 16

NPAD = 10240
CHUNK = 128
NWORK = 32
CPT = 80
EPAD = NWORK * CPT * CHUNK
BN_INV = 0.9999950000374996


def _pad_edges(edge_index):
    npad = EPAD - E
    ar = jnp.arange(npad, dtype=jnp.int32)
    pad = jnp.stack([(ar * 37) % N, N + (ar % (NPAD - N))])
    return jnp.concatenate([edge_index, pad], axis=1).reshape(2, -1, CHUNK)


def _sc_degree(edge3):
    info = plsc.get_sparse_core_info()
    NC, NS = info.num_cores, info.num_subcores
    mesh = plsc.VectorSubcoreMesh(core_axis_name="c", subcore_axis_name="s")
    rows_per_tile = NPAD // NS

    @functools.partial(
        pl.kernel, mesh=mesh,
        out_type=jax.ShapeDtypeStruct((NC, NPAD), jnp.float32),
        scratch_types=[
            pltpu.VMEM((CPT, CHUNK), jnp.int32),
            pltpu.VMEM((CHUNK,), jnp.float32),
            pltpu.VMEM((rows_per_tile,), jnp.float32),
            pltpu.VMEM_SHARED((NPAD,), jnp.float32),
            pltpu.SemaphoreType.DMA,
        ],
    )
    def k(edge_hbm, out_hbm, col_v, ones_v, zero_v, acc, ssem):
        c = lax.axis_index("c")
        s = lax.axis_index("s")
        wid = c * NS + s
        for j in range(CHUNK // 16):
            ones_v[pl.ds(j * 16, 16)] = jnp.full((16,), 1.0, jnp.float32)

        def zbody(i, _):
            zero_v[pl.ds(i * 16, 16)] = jnp.zeros((16,), jnp.float32)
            return 0
        lax.fori_loop(0, rows_per_tile // 16, zbody, 0)
        pltpu.sync_copy(zero_v, acc.at[pl.ds(s * rows_per_tile, rows_per_tile)])
        pltpu.sync_copy(edge_hbm.at[1, pl.ds(wid * CPT, CPT), :], col_v)
        plsc.subcore_barrier()

        def body(i, _):
            pltpu.async_copy(ones_v, acc.at[col_v.at[i]], ssem, add=True)
            return 0
        lax.fori_loop(0, CPT, body, 0)

        def drain(i, _):
            pltpu.make_async_copy(ones_v, acc.at[col_v.at[0]], ssem).wait()
            return 0
        lax.fori_loop(0, CPT, drain, 0)
        plsc.subcore_barrier()
        pltpu.sync_copy(acc.at[pl.ds(s * rows_per_tile, rows_per_tile)],
                        out_hbm.at[c, pl.ds(s * rows_per_tile, rows_per_tile)])

    return k(edge3)


def _sc_aggregate(edge3, y):
    info = plsc.get_sparse_core_info()
    NC, NS = info.num_cores, info.num_subcores
    mesh = plsc.VectorSubcoreMesh(core_axis_name="c", subcore_axis_name="s")
    rows_per_tile = NPAD // NS

    IB = 16

    @functools.partial(
        pl.kernel, mesh=mesh,
        out_type=jax.ShapeDtypeStruct((NC, NPAD, DIM), jnp.float32),
        scratch_types=[
            pltpu.VMEM((IB, CHUNK), jnp.int32),
            pltpu.VMEM((IB, CHUNK), jnp.int32),
            pltpu.VMEM((CHUNK, DIM), jnp.float32),
            pltpu.VMEM((CHUNK, DIM), jnp.float32),
            pltpu.VMEM_SHARED((NPAD, DIM), jnp.float32),
            pltpu.SemaphoreType.DMA,
            pltpu.SemaphoreType.DMA,
        ],
    )
    def k(edge_hbm, y_hbm, out_hbm, row_v, col_v, buf0, buf1, acc, g0, g1):
        c = lax.axis_index("c")
        s = lax.axis_index("s")
        wid = c * NS + s

        def zbody(r, _):
            for j in range(DIM // 16):
                buf0[r, pl.ds(j * 16, 16)] = jnp.zeros((16,), jnp.float32)
            return 0
        lax.fori_loop(0, CHUNK, zbody, 0)
        for b in range(rows_per_tile // CHUNK):
            pltpu.sync_copy(
                buf0,
                acc.at[pl.ds(s * rows_per_tile + b * CHUNK, CHUNK), :])
        plsc.subcore_barrier()

        def slab(sb, _):
            base = wid * CPT + sb * IB
            pltpu.sync_copy(edge_hbm.at[0, pl.ds(base, IB), :], row_v)
            pltpu.sync_copy(edge_hbm.at[1, pl.ds(base, IB), :], col_v)
            pltpu.async_copy(y_hbm.at[row_v.at[0]], buf0, g0)

            def body(i, _):
                ka = 2 * i
                pltpu.make_async_copy(y_hbm.at[row_v.at[ka]], buf0, g0).wait()
                pltpu.async_copy(y_hbm.at[row_v.at[ka + 1]], buf1, g1)
                pltpu.sync_copy(buf0, acc.at[col_v.at[ka]], add=True)
                pltpu.make_async_copy(
                    y_hbm.at[row_v.at[ka + 1]], buf1, g1).wait()
                kn = jnp.minimum(ka + 2, IB - 1)
                pltpu.async_copy(y_hbm.at[row_v.at[kn]], buf0, g0)
                pltpu.sync_copy(buf1, acc.at[col_v.at[ka + 1]], add=True)
                return 0
            lax.fori_loop(0, IB // 2, body, 0)
            pltpu.make_async_copy(y_hbm.at[row_v.at[IB - 1]], buf0, g0).wait()
            return 0
        lax.fori_loop(0, CPT // IB, slab, 0)
        plsc.subcore_barrier()
        pltpu.sync_copy(
            acc.at[pl.ds(s * rows_per_tile, rows_per_tile), :],
            out_hbm.at[c, pl.ds(s * rows_per_tile, rows_per_tile), :])

    return k(edge3, y)


_R = 2000


def _dinv(d_ref):
    return lax.rsqrt(d_ref[0] + d_ref[1] + 1.0)


def _dspec():
    return pl.BlockSpec((2, _R, 1), lambda i: (0, i, 0))


def _aspec():
    return pl.BlockSpec((2, _R, DIM), lambda i: (0, i, 0))


def _tc_first(x, W1, deg3):
    def body(x_ref, w_ref, d_ref, y_ref):
        dinv = _dinv(d_ref)
        xl = lax.dot_general(x_ref[...], w_ref[...], (((1,), (1,)), ((), ())),
                             preferred_element_type=jnp.float32)
        y_ref[...] = dinv * xl

    return pl.pallas_call(
        body,
        grid=(N // _R,),
        in_specs=[
            pl.BlockSpec((_R, DIN), lambda i: (i, 0)),
            pl.BlockSpec((DIM, DIN), lambda i: (0, 0)),
            _dspec(),
        ],
        out_specs=pl.BlockSpec((_R, DIM), lambda i: (i, 0)),
        out_shape=jax.ShapeDtypeStruct((N, DIM), jnp.float32),
    )(x, W1, deg3)


def _tc_mid(agg, y1, deg3, b1, g1, be1, W2):
    def body(a_ref, y_ref, d_ref, b_ref, g_ref, be_ref, w_ref, o_ref):
        dinv = _dinv(d_ref)
        pre = dinv * (a_ref[0] + a_ref[1] + y_ref[...]) + b_ref[...]
        h = jnp.maximum(pre, 0.0) * (g_ref[...] * BN_INV) + be_ref[...]
        hl = lax.dot_general(h, w_ref[...], (((1,), (1,)), ((), ())),
                             preferred_element_type=jnp.float32)
        o_ref[...] = dinv * hl

    return pl.pallas_call(
        body,
        grid=(N // _R,),
        in_specs=[
            _aspec(),
            pl.BlockSpec((_R, DIM), lambda i: (i, 0)),
            _dspec(),
            pl.BlockSpec((1, DIM), lambda i: (0, 0)),
            pl.BlockSpec((1, DIM), lambda i: (0, 0)),
            pl.BlockSpec((1, DIM), lambda i: (0, 0)),
            pl.BlockSpec((DIM, DIM), lambda i: (0, 0)),
        ],
        out_specs=pl.BlockSpec((_R, DIM), lambda i: (i, 0)),
        out_shape=jax.ShapeDtypeStruct((N, DIM), jnp.float32),
    )(agg, y1, deg3, b1, g1, be1, W2)


def _tc_final(agg, y2, deg3, b2, g2, be2, l1W, l1b, l2W, l2b):
    def body(a_ref, y_ref, d_ref, b_ref, g_ref, be_ref,
             w1_ref, w1b_ref, w2_ref, w2b_ref, o_ref):
        dinv = _dinv(d_ref)
        pre = dinv * (a_ref[0] + a_ref[1] + y_ref[...]) + b_ref[...]
        h = jnp.maximum(pre, 0.0) * (g_ref[...] * BN_INV) + be_ref[...]
        h = lax.dot_general(h, w1_ref[...], (((1,), (1,)), ((), ())),
                            preferred_element_type=jnp.float32) + w1b_ref[...]
        h = jnp.maximum(h, 0.0)
        lg = lax.dot_general(h, w2_ref[...], (((1,), (1,)), ((), ())),
                             preferred_element_type=jnp.float32) + w2b_ref[...]
        m = jnp.max(lg, axis=-1, keepdims=True)
        lse = jnp.log(jnp.sum(jnp.exp(lg - m), axis=-1, keepdims=True)) + m
        o_ref[...] = lg - lse

    return pl.pallas_call(
        body,
        grid=(N // _R,),
        in_specs=[
            _aspec(),
            pl.BlockSpec((_R, DIM), lambda i: (i, 0)),
            _dspec(),
            pl.BlockSpec((1, DIM), lambda i: (0, 0)),
            pl.BlockSpec((1, DIM), lambda i: (0, 0)),
            pl.BlockSpec((1, DIM), lambda i: (0, 0)),
            pl.BlockSpec((DIM, DIM), lambda i: (0, 0)),
            pl.BlockSpec((1, DIM), lambda i: (0, 0)),
            pl.BlockSpec((DOUT, DIM), lambda i: (0, 0)),
            pl.BlockSpec((1, DOUT), lambda i: (0, 0)),
        ],
        out_specs=pl.BlockSpec((_R, DOUT), lambda i: (i, 0)),
        out_shape=jax.ShapeDtypeStruct((N, DOUT), jnp.float32),
    )(agg, y2, deg3, b2, g2, be2, l1W, l1b, l2W, l2b)


def kernel(x, edge_index, batch, W1, b1, g1, be1, W2, b2, g2, be2,
           l1W, l1b, l2W, l2b):
    del batch
    edge3 = _pad_edges(edge_index.astype(jnp.int32))

    deg3 = _sc_degree(edge3).reshape(2, NPAD, 1)
    y1 = _tc_first(x, W1, deg3)
    agg1 = _sc_aggregate(edge3, y1)
    y2 = _tc_mid(agg1, y1, deg3,
                 b1.reshape(1, DIM), g1.reshape(1, DIM), be1.reshape(1, DIM),
                 W2)
    agg2 = _sc_aggregate(edge3, y2)
    return _tc_final(agg2, y2, deg3,
                     b2.reshape(1, DIM), g2.reshape(1, DIM),
                     be2.reshape(1, DIM),
                     l1W, l1b.reshape(1, DIM), l2W, l2b.reshape(1, DOUT))


# speedup vs baseline: 1.1656x; 1.0424x over previous
import functools

import jax
import jax.numpy as jnp
from jax import lax
from jax.experimental import pallas as pl
from jax.experimental.pallas import tpu as pltpu
from jax.experimental.pallas import tpu_sc as plsc

N = 10000
E = 320000
DIN = 128
DIM = 128
DOUT = 16

NPAD = 10240
CHUNK = 128
NWORK = 32
CPT = 80
EPAD = NWORK * CPT * CHUNK
BN_INV = 0.9999950000374996


def _pad_edges(edge_index):
    npad = EPAD - E
    ar = jnp.arange(npad, dtype=jnp.int32)
    pad = jnp.stack([(ar * 37) % N, N + (ar % (NPAD - N))])
    return jnp.concatenate([edge_index, pad], axis=1).reshape(2, -1, CHUNK)


def _sc_degree(edge3):
    info = plsc.get_sparse_core_info()
    NC, NS = info.num_cores, info.num_subcores
    mesh = plsc.VectorSubcoreMesh(core_axis_name="c", subcore_axis_name="s")
    rows_per_tile = NPAD // NS

    @functools.partial(
        pl.kernel, mesh=mesh,
        out_type=jax.ShapeDtypeStruct((NC, NPAD), jnp.float32),
        scratch_types=[
            pltpu.VMEM((CPT, CHUNK), jnp.int32),
            pltpu.VMEM((CHUNK,), jnp.float32),
            pltpu.VMEM((rows_per_tile,), jnp.float32),
            pltpu.VMEM_SHARED((NPAD,), jnp.float32),
            pltpu.SemaphoreType.DMA,
        ],
    )
    def k(edge_hbm, out_hbm, col_v, ones_v, zero_v, acc, ssem):
        c = lax.axis_index("c")
        s = lax.axis_index("s")
        wid = c * NS + s
        for j in range(CHUNK // 16):
            ones_v[pl.ds(j * 16, 16)] = jnp.full((16,), 1.0, jnp.float32)

        def zbody(i, _):
            zero_v[pl.ds(i * 16, 16)] = jnp.zeros((16,), jnp.float32)
            return 0
        lax.fori_loop(0, rows_per_tile // 16, zbody, 0)
        pltpu.sync_copy(zero_v, acc.at[pl.ds(s * rows_per_tile, rows_per_tile)])
        pltpu.sync_copy(edge_hbm.at[1, pl.ds(wid * CPT, CPT), :], col_v)
        plsc.subcore_barrier()

        def body(i, _):
            pltpu.async_copy(ones_v, acc.at[col_v.at[i]], ssem, add=True)
            return 0
        lax.fori_loop(0, CPT, body, 0)

        def drain(i, _):
            pltpu.make_async_copy(ones_v, acc.at[col_v.at[0]], ssem).wait()
            return 0
        lax.fori_loop(0, CPT, drain, 0)
        plsc.subcore_barrier()
        pltpu.sync_copy(acc.at[pl.ds(s * rows_per_tile, rows_per_tile)],
                        out_hbm.at[c, pl.ds(s * rows_per_tile, rows_per_tile)])

    return k(edge3)


def _sc_aggregate(edge3, y):
    info = plsc.get_sparse_core_info()
    NC, NS = info.num_cores, info.num_subcores
    mesh = plsc.VectorSubcoreMesh(core_axis_name="c", subcore_axis_name="s")
    rows_per_tile = NPAD // NS

    IB = 40

    @functools.partial(
        pl.kernel, mesh=mesh,
        out_type=jax.ShapeDtypeStruct((NC, NPAD, DIM), jnp.float32),
        scratch_types=[
            pltpu.VMEM((IB, CHUNK), jnp.int32),
            pltpu.VMEM((IB, CHUNK), jnp.int32),
            pltpu.VMEM((CHUNK, DIM), jnp.float32),
            pltpu.VMEM((CHUNK, DIM), jnp.float32),
            pltpu.VMEM_SHARED((NPAD, DIM), jnp.float32),
            pltpu.SemaphoreType.DMA,
            pltpu.SemaphoreType.DMA,
        ],
    )
    def k(edge_hbm, y_hbm, out_hbm, row_v, col_v, buf0, buf1, acc, g0, g1):
        c = lax.axis_index("c")
        s = lax.axis_index("s")
        wid = c * NS + s

        def zbody(r, _):
            for j in range(DIM // 16):
                buf0[r, pl.ds(j * 16, 16)] = jnp.zeros((16,), jnp.float32)
            return 0
        lax.fori_loop(0, CHUNK, zbody, 0)
        for b in range(rows_per_tile // CHUNK):
            pltpu.sync_copy(
                buf0,
                acc.at[pl.ds(s * rows_per_tile + b * CHUNK, CHUNK), :])
        plsc.subcore_barrier()

        def slab(sb, _):
            base = wid * CPT + sb * IB
            pltpu.sync_copy(edge_hbm.at[0, pl.ds(base, IB), :], row_v)
            pltpu.sync_copy(edge_hbm.at[1, pl.ds(base, IB), :], col_v)
            pltpu.async_copy(y_hbm.at[row_v.at[0]], buf0, g0)

            def body(i, _):
                ka = 2 * i
                pltpu.make_async_copy(y_hbm.at[row_v.at[ka]], buf0, g0).wait()
                pltpu.async_copy(y_hbm.at[row_v.at[ka + 1]], buf1, g1)
                pltpu.sync_copy(buf0, acc.at[col_v.at[ka]], add=True)
                pltpu.make_async_copy(
                    y_hbm.at[row_v.at[ka + 1]], buf1, g1).wait()
                kn = jnp.minimum(ka + 2, IB - 1)
                pltpu.async_copy(y_hbm.at[row_v.at[kn]], buf0, g0)
                pltpu.sync_copy(buf1, acc.at[col_v.at[ka + 1]], add=True)
                return 0
            lax.fori_loop(0, IB // 2, body, 0)
            pltpu.make_async_copy(y_hbm.at[row_v.at[IB - 1]], buf0, g0).wait()
            return 0
        lax.fori_loop(0, CPT // IB, slab, 0)
        plsc.subcore_barrier()
        pltpu.sync_copy(
            acc.at[pl.ds(s * rows_per_tile, rows_per_tile), :],
            out_hbm.at[c, pl.ds(s * rows_per_tile, rows_per_tile), :])

    return k(edge3, y)


_R = 2000


def _dinv(d_ref):
    return lax.rsqrt(d_ref[0] + d_ref[1] + 1.0)


def _dspec():
    return pl.BlockSpec((2, _R, 1), lambda i: (0, i, 0))


def _aspec():
    return pl.BlockSpec((2, _R, DIM), lambda i: (0, i, 0))


def _tc_first(x, W1, deg3):
    def body(x_ref, w_ref, d_ref, y_ref):
        dinv = _dinv(d_ref)
        xl = lax.dot_general(x_ref[...], w_ref[...], (((1,), (1,)), ((), ())),
                             preferred_element_type=jnp.float32)
        y_ref[...] = dinv * xl

    return pl.pallas_call(
        body,
        grid=(N // _R,),
        in_specs=[
            pl.BlockSpec((_R, DIN), lambda i: (i, 0)),
            pl.BlockSpec((DIM, DIN), lambda i: (0, 0)),
            _dspec(),
        ],
        out_specs=pl.BlockSpec((_R, DIM), lambda i: (i, 0)),
        out_shape=jax.ShapeDtypeStruct((N, DIM), jnp.float32),
    )(x, W1, deg3)


def _tc_mid(agg, y1, deg3, b1, g1, be1, W2):
    def body(a_ref, y_ref, d_ref, b_ref, g_ref, be_ref, w_ref, o_ref):
        dinv = _dinv(d_ref)
        pre = dinv * (a_ref[0] + a_ref[1] + y_ref[...]) + b_ref[...]
        h = jnp.maximum(pre, 0.0) * (g_ref[...] * BN_INV) + be_ref[...]
        hl = lax.dot_general(h, w_ref[...], (((1,), (1,)), ((), ())),
                             preferred_element_type=jnp.float32)
        o_ref[...] = dinv * hl

    return pl.pallas_call(
        body,
        grid=(N // _R,),
        in_specs=[
            _aspec(),
            pl.BlockSpec((_R, DIM), lambda i: (i, 0)),
            _dspec(),
            pl.BlockSpec((1, DIM), lambda i: (0, 0)),
            pl.BlockSpec((1, DIM), lambda i: (0, 0)),
            pl.BlockSpec((1, DIM), lambda i: (0, 0)),
            pl.BlockSpec((DIM, DIM), lambda i: (0, 0)),
        ],
        out_specs=pl.BlockSpec((_R, DIM), lambda i: (i, 0)),
        out_shape=jax.ShapeDtypeStruct((N, DIM), jnp.float32),
    )(agg, y1, deg3, b1, g1, be1, W2)


def _tc_final(agg, y2, deg3, b2, g2, be2, l1W, l1b, l2W, l2b):
    def body(a_ref, y_ref, d_ref, b_ref, g_ref, be_ref,
             w1_ref, w1b_ref, w2_ref, w2b_ref, o_ref):
        dinv = _dinv(d_ref)
        pre = dinv * (a_ref[0] + a_ref[1] + y_ref[...]) + b_ref[...]
        h = jnp.maximum(pre, 0.0) * (g_ref[...] * BN_INV) + be_ref[...]
        h = lax.dot_general(h, w1_ref[...], (((1,), (1,)), ((), ())),
                            preferred_element_type=jnp.float32) + w1b_ref[...]
        h = jnp.maximum(h, 0.0)
        lg = lax.dot_general(h, w2_ref[...], (((1,), (1,)), ((), ())),
                             preferred_element_type=jnp.float32) + w2b_ref[...]
        m = jnp.max(lg, axis=-1, keepdims=True)
        lse = jnp.log(jnp.sum(jnp.exp(lg - m), axis=-1, keepdims=True)) + m
        o_ref[...] = lg - lse

    return pl.pallas_call(
        body,
        grid=(N // _R,),
        in_specs=[
            _aspec(),
            pl.BlockSpec((_R, DIM), lambda i: (i, 0)),
            _dspec(),
            pl.BlockSpec((1, DIM), lambda i: (0, 0)),
            pl.BlockSpec((1, DIM), lambda i: (0, 0)),
            pl.BlockSpec((1, DIM), lambda i: (0, 0)),
            pl.BlockSpec((DIM, DIM), lambda i: (0, 0)),
            pl.BlockSpec((1, DIM), lambda i: (0, 0)),
            pl.BlockSpec((DOUT, DIM), lambda i: (0, 0)),
            pl.BlockSpec((1, DOUT), lambda i: (0, 0)),
        ],
        out_specs=pl.BlockSpec((_R, DOUT), lambda i: (i, 0)),
        out_shape=jax.ShapeDtypeStruct((N, DOUT), jnp.float32),
    )(agg, y2, deg3, b2, g2, be2, l1W, l1b, l2W, l2b)


def kernel(x, edge_index, batch, W1, b1, g1, be1, W2, b2, g2, be2,
           l1W, l1b, l2W, l2b):
    del batch
    edge3 = _pad_edges(edge_index.astype(jnp.int32))

    deg3 = _sc_degree(edge3).reshape(2, NPAD, 1)
    y1 = _tc_first(x, W1, deg3)
    agg1 = _sc_aggregate(edge3, y1)
    y2 = _tc_mid(agg1, y1, deg3,
                 b1.reshape(1, DIM), g1.reshape(1, DIM), be1.reshape(1, DIM),
                 W2)
    agg2 = _sc_aggregate(edge3, y2)
    return _tc_final(agg2, y2, deg3,
                     b2.reshape(1, DIM), g2.reshape(1, DIM),
                     be2.reshape(1, DIM),
                     l1W, l1b.reshape(1, DIM), l2W, l2b.reshape(1, DOUT))


# speedup vs baseline: 1.3261x; 1.1377x over previous
import functools

import jax
import jax.numpy as jnp
from jax import lax
from jax.experimental import pallas as pl
from jax.experimental.pallas import tpu as pltpu
from jax.experimental.pallas import tpu_sc as plsc

N = 10000
E = 320000
DIN = 128
DIM = 128
DOUT = 16

NPAD = 10240
CHUNK = 128
NWORK = 32
CPT = 80
EPAD = NWORK * CPT * CHUNK
BN_INV = 0.9999950000374996


def _pad_edges(edge_index):
    npad = EPAD - E
    ar = jnp.arange(npad, dtype=jnp.int32)
    pad = jnp.stack([(ar * 37) % N, N + (ar % (NPAD - N))])
    return jnp.concatenate([edge_index, pad], axis=1).reshape(2, -1, CHUNK)


def _sc_degree(edge3):
    info = plsc.get_sparse_core_info()
    NC, NS = info.num_cores, info.num_subcores
    mesh = plsc.VectorSubcoreMesh(core_axis_name="c", subcore_axis_name="s")
    rows_per_tile = NPAD // NS

    @functools.partial(
        pl.kernel, mesh=mesh,
        out_type=jax.ShapeDtypeStruct((NC, NPAD), jnp.float32),
        scratch_types=[
            pltpu.VMEM((CPT, CHUNK), jnp.int32),
            pltpu.VMEM((CHUNK,), jnp.float32),
            pltpu.VMEM((rows_per_tile,), jnp.float32),
            pltpu.VMEM_SHARED((NPAD,), jnp.float32),
            pltpu.SemaphoreType.DMA,
        ],
    )
    def k(edge_hbm, out_hbm, col_v, ones_v, zero_v, acc, ssem):
        c = lax.axis_index("c")
        s = lax.axis_index("s")
        wid = c * NS + s
        for j in range(CHUNK // 16):
            ones_v[pl.ds(j * 16, 16)] = jnp.full((16,), 1.0, jnp.float32)

        def zbody(i, _):
            zero_v[pl.ds(i * 16, 16)] = jnp.zeros((16,), jnp.float32)
            return 0
        lax.fori_loop(0, rows_per_tile // 16, zbody, 0)
        pltpu.sync_copy(zero_v, acc.at[pl.ds(s * rows_per_tile, rows_per_tile)])
        pltpu.sync_copy(edge_hbm.at[1, pl.ds(wid * CPT, CPT), :], col_v)
        plsc.subcore_barrier()

        def body(i, _):
            pltpu.async_copy(ones_v, acc.at[col_v.at[i]], ssem, add=True)
            return 0
        lax.fori_loop(0, CPT, body, 0)

        def drain(i, _):
            pltpu.make_async_copy(ones_v, acc.at[col_v.at[0]], ssem).wait()
            return 0
        lax.fori_loop(0, CPT, drain, 0)
        plsc.subcore_barrier()
        pltpu.sync_copy(acc.at[pl.ds(s * rows_per_tile, rows_per_tile)],
                        out_hbm.at[c, pl.ds(s * rows_per_tile, rows_per_tile)])

    return k(edge3)


def _sc_aggregate(edge3, y):
    info = plsc.get_sparse_core_info()
    NC, NS = info.num_cores, info.num_subcores
    mesh = plsc.VectorSubcoreMesh(core_axis_name="c", subcore_axis_name="s")
    rows_per_tile = NPAD // NS

    IB = 40

    @functools.partial(
        pl.kernel, mesh=mesh,
        out_type=jax.ShapeDtypeStruct((NC, NPAD, DIM), jnp.float32),
        scratch_types=[
            pltpu.VMEM((IB, CHUNK), jnp.int32),
            pltpu.VMEM((IB, CHUNK), jnp.int32),
            pltpu.VMEM((CHUNK, DIM), jnp.float32),
            pltpu.VMEM((CHUNK, DIM), jnp.float32),
            pltpu.VMEM_SHARED((NPAD, DIM), jnp.float32),
            pltpu.SemaphoreType.DMA,
            pltpu.SemaphoreType.DMA,
        ],
    )
    def k(edge_hbm, y_hbm, out_hbm, row_v, col_v, buf0, buf1, acc, g0, g1):
        c = lax.axis_index("c")
        s = lax.axis_index("s")
        wid = c * NS + s

        def zbody(r, _):
            for j in range(DIM // 16):
                buf0[r, pl.ds(j * 16, 16)] = jnp.zeros((16,), jnp.float32)
            return 0
        lax.fori_loop(0, CHUNK, zbody, 0)
        for b in range(rows_per_tile // CHUNK):
            pltpu.sync_copy(
                buf0,
                acc.at[pl.ds(s * rows_per_tile + b * CHUNK, CHUNK), :])
        plsc.subcore_barrier()

        def slab(sb, _):
            base = wid * CPT + sb * IB
            pltpu.sync_copy(edge_hbm.at[0, pl.ds(base, IB), :], row_v)
            pltpu.sync_copy(edge_hbm.at[1, pl.ds(base, IB), :], col_v)
            pltpu.async_copy(y_hbm.at[row_v.at[0]], buf0, g0)
            pltpu.async_copy(y_hbm.at[row_v.at[1]], buf1, g1)

            def body(i, _):
                ka = 2 * i
                pltpu.make_async_copy(y_hbm.at[row_v.at[ka]], buf0, g0).wait()
                pltpu.sync_copy(buf0, acc.at[col_v.at[ka]], add=True)
                kn = jnp.minimum(ka + 2, IB - 1)
                pltpu.async_copy(y_hbm.at[row_v.at[kn]], buf0, g0)
                pltpu.make_async_copy(
                    y_hbm.at[row_v.at[ka + 1]], buf1, g1).wait()
                pltpu.sync_copy(buf1, acc.at[col_v.at[ka + 1]], add=True)
                km = jnp.minimum(ka + 3, IB - 1)
                pltpu.async_copy(y_hbm.at[row_v.at[km]], buf1, g1)
                return 0
            lax.fori_loop(0, IB // 2, body, 0)
            pltpu.make_async_copy(y_hbm.at[row_v.at[IB - 1]], buf0, g0).wait()
            pltpu.make_async_copy(y_hbm.at[row_v.at[IB - 1]], buf1, g1).wait()
            return 0
        lax.fori_loop(0, CPT // IB, slab, 0)
        plsc.subcore_barrier()
        pltpu.sync_copy(
            acc.at[pl.ds(s * rows_per_tile, rows_per_tile), :],
            out_hbm.at[c, pl.ds(s * rows_per_tile, rows_per_tile), :])

    return k(edge3, y)


_R = 2000


def _dinv(d_ref):
    return lax.rsqrt(d_ref[0] + d_ref[1] + 1.0)


def _dspec():
    return pl.BlockSpec((2, _R, 1), lambda i: (0, i, 0))


def _aspec():
    return pl.BlockSpec((2, _R, DIM), lambda i: (0, i, 0))


def _tc_first(x, W1, deg3):
    def body(x_ref, w_ref, d_ref, y_ref):
        dinv = _dinv(d_ref)
        xl = lax.dot_general(x_ref[...], w_ref[...], (((1,), (1,)), ((), ())),
                             preferred_element_type=jnp.float32)
        y_ref[...] = dinv * xl

    return pl.pallas_call(
        body,
        grid=(N // _R,),
        in_specs=[
            pl.BlockSpec((_R, DIN), lambda i: (i, 0)),
            pl.BlockSpec((DIM, DIN), lambda i: (0, 0)),
            _dspec(),
        ],
        out_specs=pl.BlockSpec((_R, DIM), lambda i: (i, 0)),
        out_shape=jax.ShapeDtypeStruct((N, DIM), jnp.float32),
    )(x, W1, deg3)


def _tc_mid(agg, y1, deg3, b1, g1, be1, W2):
    def body(a_ref, y_ref, d_ref, b_ref, g_ref, be_ref, w_ref, o_ref):
        dinv = _dinv(d_ref)
        pre = dinv * (a_ref[0] + a_ref[1] + y_ref[...]) + b_ref[...]
        h = jnp.maximum(pre, 0.0) * (g_ref[...] * BN_INV) + be_ref[...]
        hl = lax.dot_general(h, w_ref[...], (((1,), (1,)), ((), ())),
                             preferred_element_type=jnp.float32)
        o_ref[...] = dinv * hl

    return pl.pallas_call(
        body,
        grid=(N // _R,),
        in_specs=[
            _aspec(),
            pl.BlockSpec((_R, DIM), lambda i: (i, 0)),
            _dspec(),
            pl.BlockSpec((1, DIM), lambda i: (0, 0)),
            pl.BlockSpec((1, DIM), lambda i: (0, 0)),
            pl.BlockSpec((1, DIM), lambda i: (0, 0)),
            pl.BlockSpec((DIM, DIM), lambda i: (0, 0)),
        ],
        out_specs=pl.BlockSpec((_R, DIM), lambda i: (i, 0)),
        out_shape=jax.ShapeDtypeStruct((N, DIM), jnp.float32),
    )(agg, y1, deg3, b1, g1, be1, W2)


def _tc_final(agg, y2, deg3, b2, g2, be2, l1W, l1b, l2W, l2b):
    def body(a_ref, y_ref, d_ref, b_ref, g_ref, be_ref,
             w1_ref, w1b_ref, w2_ref, w2b_ref, o_ref):
        dinv = _dinv(d_ref)
        pre = dinv * (a_ref[0] + a_ref[1] + y_ref[...]) + b_ref[...]
        h = jnp.maximum(pre, 0.0) * (g_ref[...] * BN_INV) + be_ref[...]
        h = lax.dot_general(h, w1_ref[...], (((1,), (1,)), ((), ())),
                            preferred_element_type=jnp.float32) + w1b_ref[...]
        h = jnp.maximum(h, 0.0)
        lg = lax.dot_general(h, w2_ref[...], (((1,), (1,)), ((), ())),
                             preferred_element_type=jnp.float32) + w2b_ref[...]
        m = jnp.max(lg, axis=-1, keepdims=True)
        lse = jnp.log(jnp.sum(jnp.exp(lg - m), axis=-1, keepdims=True)) + m
        o_ref[...] = lg - lse

    return pl.pallas_call(
        body,
        grid=(N // _R,),
        in_specs=[
            _aspec(),
            pl.BlockSpec((_R, DIM), lambda i: (i, 0)),
            _dspec(),
            pl.BlockSpec((1, DIM), lambda i: (0, 0)),
            pl.BlockSpec((1, DIM), lambda i: (0, 0)),
            pl.BlockSpec((1, DIM), lambda i: (0, 0)),
            pl.BlockSpec((DIM, DIM), lambda i: (0, 0)),
            pl.BlockSpec((1, DIM), lambda i: (0, 0)),
            pl.BlockSpec((DOUT, DIM), lambda i: (0, 0)),
            pl.BlockSpec((1, DOUT), lambda i: (0, 0)),
        ],
        out_specs=pl.BlockSpec((_R, DOUT), lambda i: (i, 0)),
        out_shape=jax.ShapeDtypeStruct((N, DOUT), jnp.float32),
    )(agg, y2, deg3, b2, g2, be2, l1W, l1b, l2W, l2b)


def kernel(x, edge_index, batch, W1, b1, g1, be1, W2, b2, g2, be2,
           l1W, l1b, l2W, l2b):
    del batch
    edge3 = _pad_edges(edge_index.astype(jnp.int32))

    deg3 = _sc_degree(edge3).reshape(2, NPAD, 1)
    y1 = _tc_first(x, W1, deg3)
    agg1 = _sc_aggregate(edge3, y1)
    y2 = _tc_mid(agg1, y1, deg3,
                 b1.reshape(1, DIM), g1.reshape(1, DIM), be1.reshape(1, DIM),
                 W2)
    agg2 = _sc_aggregate(edge3, y2)
    return _tc_final(agg2, y2, deg3,
                     b2.reshape(1, DIM), g2.reshape(1, DIM),
                     be2.reshape(1, DIM),
                     l1W, l1b.reshape(1, DIM), l2W, l2b.reshape(1, DOUT))
